# baseline (speedup 1.0000x reference)
import functools

import jax
import jax.numpy as jnp
from jax import lax
from jax.experimental import pallas as pl
from jax.experimental.pallas import tpu as pltpu
from jax.experimental.pallas import tpu_sc as plsc

N = 10000
D = 128
OUT = 128
HD = D // 2
NC = 2
NS = 16
L = 16
B = 128
NPAD = 10112


def _sc_agg(nblk):
    mesh = plsc.VectorSubcoreMesh(core_axis_name="c", subcore_axis_name="s")

    @functools.partial(
        pl.kernel,
        out_type=jax.ShapeDtypeStruct((NC, NPAD, HD), jnp.float32),
        mesh=mesh,
        scratch_types=[
            pltpu.VMEM((nblk, B), jnp.int32),
            pltpu.VMEM((nblk, B), jnp.int32),
            pltpu.VMEM((nblk, B), jnp.float32),
            pltpu.VMEM((B, HD), jnp.float32),
            pltpu.VMEM_SHARED((NPAD, HD), jnp.float32),
            pltpu.SemaphoreType.DMA,
        ],
        compiler_params=pltpu.CompilerParams(use_tc_tiling_on_sc=False),
    )
    def agg(nf_hbm, src_hbm, dst_hbm, w_hbm, out_hbm,
            src_v, dst_v, w_v, rows, acc, sem):
        c = lax.axis_index("c")
        s = lax.axis_index("s")

        pltpu.sync_copy(src_hbm.at[s], src_v)
        pltpu.sync_copy(dst_hbm.at[s], dst_v)
        pltpu.sync_copy(w_hbm.at[s], w_v)

        cofs = jnp.full((L,), c * N, jnp.int32)

        def ofs_body(i, _):
            for j in range(B // L):
                sl = pl.ds(j * L, L)
                src_v[i, sl] = src_v[i, sl] + cofs
            return 0
        lax.fori_loop(0, nblk, ofs_body, 0)

        def zrow(i, _):
            for j in range(HD // L):
                rows[i, pl.ds(j * L, L)] = jnp.zeros((L,), jnp.float32)
            return 0
        lax.fori_loop(0, B, zrow, 0)
        stripe = NPAD // NS
        base = s * stripe
        for t in range(stripe // B):
            pltpu.sync_copy(rows, acc.at[pl.ds(base + t * B, B)])
        rem = stripe - (stripe // B) * B
        if rem:
            pltpu.sync_copy(rows.at[pl.ds(0, rem)],
                            acc.at[pl.ds(base + stripe - rem, rem)])
        plsc.subcore_barrier()

        def blk_body(blk, _):
            pltpu.async_copy(nf_hbm.at[src_v.at[blk]], rows, sem).wait()

            @plsc.parallel_loop(0, B, step=1, unroll=8)
            def _(e):
                gbase = (e // L) * L
                wg = w_v[blk, pl.ds(gbase, L)]
                lane = e - gbase
                wv = wg.at[jnp.full((L,), lane, jnp.int32)].get(
                    mode='promise_in_bounds')
                for j in range(HD // L):
                    sl = pl.ds(j * L, L)
                    rows[e, sl] = rows[e, sl] * wv

            pltpu.sync_copy(rows, acc.at[dst_v.at[blk]], add=True)
            return 0
        lax.fori_loop(0, nblk, blk_body, 0)

        plsc.subcore_barrier()
        pltpu.sync_copy(acc.at[pl.ds(base, stripe)],
                        out_hbm.at[c, pl.ds(base, stripe)])

    return agg


def _combine_body(p_ref, w_ref, b_ref, o_ref):
    o_ref[...] = (
        jnp.dot(p_ref[0], w_ref[:HD, :], preferred_element_type=jnp.float32)
        + jnp.dot(p_ref[1], w_ref[HD:, :], preferred_element_type=jnp.float32)
        + b_ref[...]
    )


@jax.jit
def kernel(node_features, edge_index, edge_weight, W, b):
    E = edge_weight.shape[0]
    ept = -(-E // NS)
    nblk = -(-ept // B)
    ept = nblk * B
    pad = ept * NS - E

    src = jnp.pad(edge_index[1], (0, pad))
    dst = jnp.pad(edge_index[0], (0, pad))
    w = jnp.pad(edge_weight, (0, pad))

    srcb = src.reshape(NS, nblk, B)
    dstb = dst.reshape(NS, nblk, B)
    wb = w.reshape(NS, nblk, B)

    nfh = node_features.reshape(N, NC, HD).transpose(1, 0, 2)
    nfh = nfh.reshape(NC * N, HD)

    partials = _sc_agg(nblk)(nfh, srcb, dstb, wb)

    BM = 1000
    out = pl.pallas_call(
        _combine_body,
        grid=(N // BM,),
        in_specs=[
            pl.BlockSpec((NC, BM, HD), lambda i: (0, i, 0)),
            pl.BlockSpec((D, OUT), lambda i: (0, 0)),
            pl.BlockSpec((1, OUT), lambda i: (0, 0)),
        ],
        out_specs=pl.BlockSpec((BM, OUT), lambda i: (i, 0)),
        out_shape=jax.ShapeDtypeStruct((N, OUT), jnp.float32),
    )(partials, W, b.reshape(1, OUT))
    return out


# speedup vs baseline: 1.1660x; 1.1660x over previous
import functools

import jax
import jax.numpy as jnp
from jax import lax
from jax.experimental import pallas as pl
from jax.experimental.pallas import tpu as pltpu
from jax.experimental.pallas import tpu_sc as plsc

N = 10000
D = 128
OUT = 128
PW = D // 2
NC = 2
NS = 16
L = 16
NW = NC * NS
B = 96
NPAD = 10112


def _sc_agg(nblk):
    mesh = plsc.VectorSubcoreMesh(core_axis_name="c", subcore_axis_name="s")

    @functools.partial(
        pl.kernel,
        out_type=jax.ShapeDtypeStruct((NC, NPAD, D), jnp.float32),
        mesh=mesh,
        scratch_types=[
            pltpu.VMEM((nblk, B), jnp.int32),
            pltpu.VMEM((nblk, B), jnp.int32),
            pltpu.VMEM((nblk, B), jnp.float32),
            pltpu.VMEM((B, PW), jnp.int32),
            pltpu.VMEM((B, D), jnp.float32),
            pltpu.VMEM_SHARED((NPAD, D), jnp.float32),
            pltpu.SemaphoreType.DMA,
        ],
        compiler_params=pltpu.CompilerParams(
            use_tc_tiling_on_sc=False, needs_layout_passes=False),
    )
    def agg(nf_hbm, src_hbm, dst_hbm, w_hbm, out_hbm,
            src_v, dst_v, w_v, rows, scaled, acc, sem):
        c = lax.axis_index("c")
        s = lax.axis_index("s")
        wid = s * NC + c

        pltpu.sync_copy(src_hbm.at[wid], src_v)
        pltpu.sync_copy(dst_hbm.at[wid], dst_v)
        pltpu.sync_copy(w_hbm.at[wid], w_v)

        def zrow(i, _):
            for j in range(D // L):
                scaled[i, pl.ds(j * L, L)] = jnp.zeros((L,), jnp.float32)
            return 0
        lax.fori_loop(0, B, zrow, 0)
        stripe = NPAD // NS
        base = s * stripe
        for t in range(stripe // B):
            pltpu.sync_copy(scaled, acc.at[pl.ds(base + t * B, B)])
        rem = stripe - (stripe // B) * B
        if rem:
            pltpu.sync_copy(scaled.at[pl.ds(0, rem)],
                            acc.at[pl.ds(base + stripe - rem, rem)])
        plsc.subcore_barrier()

        def blk_body(blk, _):
            pltpu.async_copy(nf_hbm.at[src_v.at[blk]], rows, sem).wait()

            @plsc.parallel_loop(0, B, step=1, unroll=8)
            def _(e):
                gbase = (e // L) * L
                wg = w_v[blk, pl.ds(gbase, L)]
                lane = e - gbase
                wv = wg.at[jnp.full((L,), lane, jnp.int32)].get(
                    mode='promise_in_bounds')
                for j in range(D // (2 * L)):
                    pk = rows[e, pl.ds(j * L, L)]
                    a = plsc.bitcast(pk << 16, jnp.float32)
                    bb = plsc.bitcast(pk & jnp.int32(-65536), jnp.float32)
                    scaled[e, pl.ds(j * 2 * L, L)] = a * wv
                    scaled[e, pl.ds(j * 2 * L + L, L)] = bb * wv

            pltpu.sync_copy(scaled, acc.at[dst_v.at[blk]], add=True)
            return 0
        lax.fori_loop(0, nblk, blk_body, 0)

        plsc.subcore_barrier()
        pltpu.sync_copy(acc.at[pl.ds(base, stripe)],
                        out_hbm.at[c, pl.ds(base, stripe)])

    return agg


def _combine_body(p_ref, w_ref, b_ref, o_ref):
    p = p_ref[0, :, :] + p_ref[1, :, :]
    o_ref[...] = (
        jnp.dot(p, w_ref[...], preferred_element_type=jnp.float32)
        + b_ref[...]
    )


@jax.jit
def kernel(node_features, edge_index, edge_weight, W, b):
    E = edge_weight.shape[0]
    nblk = -(-(-(-E // NW)) // B)
    epw = nblk * B
    pad = epw * NW - E

    src = jnp.pad(edge_index[1], (0, pad))
    dst = jnp.pad(edge_index[0], (0, pad))
    w = jnp.pad(edge_weight, (0, pad))

    srcb = src.reshape(NW, nblk, B)
    dstb = dst.reshape(NW, nblk, B)
    wb = w.reshape(NW, nblk, B)

    nfb = node_features.astype(jnp.bfloat16)
    nfb = nfb.reshape(N, D // 32, 2, L).transpose(0, 1, 3, 2)
    nfp = lax.bitcast_convert_type(nfb, jnp.int32).reshape(N, PW)

    partials = _sc_agg(nblk)(nfp, srcb, dstb, wb)

    BM = 1000
    out = pl.pallas_call(
        _combine_body,
        grid=(N // BM,),
        in_specs=[
            pl.BlockSpec((NC, BM, D), lambda i: (0, i, 0)),
            pl.BlockSpec((D, OUT), lambda i: (0, 0)),
            pl.BlockSpec((1, OUT), lambda i: (0, 0)),
        ],
        out_specs=pl.BlockSpec((BM, OUT), lambda i: (i, 0)),
        out_shape=jax.ShapeDtypeStruct((N, OUT), jnp.float32),
    )(partials, W, b.reshape(1, OUT))
    return out


# speedup vs baseline: 1.5440x; 1.3242x over previous
import functools

import jax
import jax.numpy as jnp
from jax import lax
from jax.experimental import pallas as pl
from jax.experimental.pallas import tpu as pltpu
from jax.experimental.pallas import tpu_sc as plsc

N = 10000
D = 128
OUT = 128
PW = D // 2
NC = 2
NS = 16
L = 16
NW = NC * NS
B = 64
NPAD = 10112


def _sc_agg(nblk):
    mesh = plsc.VectorSubcoreMesh(core_axis_name="c", subcore_axis_name="s")

    @functools.partial(
        pl.kernel,
        out_type=jax.ShapeDtypeStruct((NC, NPAD, D), jnp.float32),
        mesh=mesh,
        scratch_types=[
            pltpu.VMEM((nblk, B), jnp.int32),
            pltpu.VMEM((nblk, B), jnp.int32),
            pltpu.VMEM((nblk, B), jnp.float32),
            pltpu.VMEM((B, PW), jnp.int32),
            pltpu.VMEM((B, PW), jnp.int32),
            pltpu.VMEM((B, D), jnp.float32),
            pltpu.VMEM_SHARED((NPAD, D), jnp.float32),
            pltpu.SemaphoreType.DMA,
            pltpu.SemaphoreType.DMA,
        ],
        compiler_params=pltpu.CompilerParams(
            use_tc_tiling_on_sc=False, needs_layout_passes=False),
    )
    def agg(nf_hbm, src_hbm, dst_hbm, w_hbm, out_hbm,
            src_v, dst_v, w_v, rows0, rows1, scaled, acc, sem0, sem1):
        c = lax.axis_index("c")
        s = lax.axis_index("s")
        wid = s * NC + c

        pltpu.sync_copy(src_hbm.at[wid], src_v)
        pltpu.sync_copy(dst_hbm.at[wid], dst_v)
        pltpu.sync_copy(w_hbm.at[wid], w_v)

        def zrow(i, _):
            for j in range(D // L):
                scaled[i, pl.ds(j * L, L)] = jnp.zeros((L,), jnp.float32)
            return 0
        lax.fori_loop(0, B, zrow, 0)
        stripe = NPAD // NS
        base = s * stripe
        for t in range(stripe // B):
            pltpu.sync_copy(scaled, acc.at[pl.ds(base + t * B, B)])
        rem = stripe - (stripe // B) * B
        if rem:
            pltpu.sync_copy(scaled.at[pl.ds(0, rem)],
                            acc.at[pl.ds(base + stripe - rem, rem)])
        plsc.subcore_barrier()

        pltpu.async_copy(nf_hbm.at[src_v.at[0]], rows0, sem0)
        pltpu.async_copy(nf_hbm.at[src_v.at[1]], rows1, sem1)

        def half_step(blk, rows, sem):
            pltpu.make_async_copy(
                nf_hbm.at[src_v.at[blk]], rows, sem).wait()

            @plsc.parallel_loop(0, B, step=1, unroll=8)
            def _(e):
                gbase = (e // L) * L
                wg = w_v[blk, pl.ds(gbase, L)]
                lane = e - gbase
                wv = wg.at[jnp.full((L,), lane, jnp.int32)].get(
                    mode='promise_in_bounds')
                for j in range(D // (2 * L)):
                    pk = rows[e, pl.ds(j * L, L)]
                    a = plsc.bitcast(pk << 16, jnp.float32)
                    bb = plsc.bitcast(pk & jnp.int32(-65536), jnp.float32)
                    scaled[e, pl.ds(j * 2 * L, L)] = a * wv
                    scaled[e, pl.ds(j * 2 * L + L, L)] = bb * wv

            @pl.when(blk + 2 < nblk)
            def _():
                pltpu.async_copy(nf_hbm.at[src_v.at[blk + 2]], rows, sem)

            pltpu.sync_copy(scaled, acc.at[dst_v.at[blk]], add=True)

        def blk_body(o, _):
            half_step(o * 2, rows0, sem0)
            half_step(o * 2 + 1, rows1, sem1)
            return 0
        lax.fori_loop(0, nblk // 2, blk_body, 0)

        plsc.subcore_barrier()
        pltpu.sync_copy(acc.at[pl.ds(base, stripe)],
                        out_hbm.at[c, pl.ds(base, stripe)])

    return agg


def _combine_body(p_ref, w_ref, b_ref, o_ref):
    p = p_ref[0, :, :] + p_ref[1, :, :]
    o_ref[...] = (
        jnp.dot(p, w_ref[...], preferred_element_type=jnp.float32)
        + b_ref[...]
    )


@jax.jit
def kernel(node_features, edge_index, edge_weight, W, b):
    E = edge_weight.shape[0]
    nblk = -(-(-(-E // NW)) // B)
    nblk = -(-nblk // 2) * 2
    epw = nblk * B
    pad = epw * NW - E

    src = jnp.pad(edge_index[1], (0, pad))
    dst = jnp.pad(edge_index[0], (0, pad))
    w = jnp.pad(edge_weight, (0, pad))

    srcb = src.reshape(NW, nblk, B)
    dstb = dst.reshape(NW, nblk, B)
    wb = w.reshape(NW, nblk, B)

    nfb = node_features.astype(jnp.bfloat16)
    nfb = nfb.reshape(N, D // 32, 2, L).transpose(0, 1, 3, 2)
    nfp = lax.bitcast_convert_type(nfb, jnp.int32).reshape(N, PW)

    partials = _sc_agg(nblk)(nfp, srcb, dstb, wb)

    BM = 1000
    out = pl.pallas_call(
        _combine_body,
        grid=(N // BM,),
        in_specs=[
            pl.BlockSpec((NC, BM, D), lambda i: (0, i, 0)),
            pl.BlockSpec((D, OUT), lambda i: (0, 0)),
            pl.BlockSpec((1, OUT), lambda i: (0, 0)),
        ],
        out_specs=pl.BlockSpec((BM, OUT), lambda i: (i, 0)),
        out_shape=jax.ShapeDtypeStruct((N, OUT), jnp.float32),
    )(partials, W, b.reshape(1, OUT))
    return out
